# baseline (speedup 1.0000x reference)
import dataclasses
import functools

import jax
import jax.numpy as jnp
from jax import lax
from jax.experimental import pallas as pl
from jax.experimental.pallas import tpu as pltpu
from jax.experimental.pallas import tpu_sc as plsc

N = 10000
D = 128
E = 320000
NCLS = 40

NC = 2
NS = 16
NW = NC * NS
L = 16

EPAD = 327680
EPT = EPAD // NW
C = 80
NWIN = EPT // C
RB = 4
IB = 8
NPAD = 10240
RPT = NPAD // NS


def _encode(x, W_enc, b_enc, W1):
    def body(x_ref, we_ref, be_ref, w1_ref, o_ref):
        h = jnp.tanh(
            jnp.dot(x_ref[...], we_ref[...], preferred_element_type=jnp.float32)
            + be_ref[...]
        )
        o_ref[...] = jnp.dot(h, w1_ref[...], preferred_element_type=jnp.float32)

    return pl.pallas_call(
        body,
        out_shape=jax.ShapeDtypeStruct((N, D), jnp.float32),
    )(x, W_enc, b_enc.reshape(1, D), W1)


def _mid(parts, b, W):
    def body(p_ref, b_ref, w_ref, o_ref):
        h = jnp.tanh(p_ref[0, :N, :] + p_ref[1, :N, :] + b_ref[...])
        o_ref[...] = jnp.dot(h, w_ref[...], preferred_element_type=jnp.float32)

    return pl.pallas_call(
        body,
        out_shape=jax.ShapeDtypeStruct((N, D), jnp.float32),
    )(parts, b.reshape(1, D), W)


def _decode(parts, b2, W_dec, b_dec):
    def body(p_ref, b2_ref, wd_ref, bd_ref, o_ref):
        h = jnp.tanh(p_ref[0, :N, :] + p_ref[1, :N, :] + b2_ref[...])
        o_ref[...] = (
            jnp.dot(h, wd_ref[...], preferred_element_type=jnp.float32)
            + bd_ref[...]
        )

    return pl.pallas_call(
        body,
        out_shape=jax.ShapeDtypeStruct((N, NCLS), jnp.float32),
    )(parts, b2.reshape(1, D), W_dec, b_dec.reshape(1, NCLS))


def _sc_edge_pass(hw, src, dst, ew, zeros):
    mesh = plsc.VectorSubcoreMesh(core_axis_name="c", subcore_axis_name="s")
    cp = pltpu.CompilerParams()
    if "needs_layout_passes" in pltpu.CompilerParams.__dataclass_fields__:
        cp = dataclasses.replace(cp, needs_layout_passes=False)

    @functools.partial(
        pl.kernel,
        mesh=mesh,
        compiler_params=cp,
        out_type=jax.ShapeDtypeStruct((NC, NPAD, D), jnp.float32),
        scratch_types=(
            [pltpu.VMEM((C,), jnp.int32) for _ in range(IB)]
            + [pltpu.VMEM((C,), jnp.int32) for _ in range(IB)]
            + [pltpu.VMEM((C,), jnp.float32) for _ in range(IB)]
            + [pltpu.VMEM((C, D), jnp.float32) for _ in range(RB)]
            + [pltpu.VMEM_SHARED((NPAD, D), jnp.float32)]
            + [pltpu.SemaphoreType.DMA for _ in range(IB + 2 * RB)]
        ),
    )
    def k(hw_hbm, src_hbm, dst_hbm, ew_hbm, z_hbm, out_hbm, *refs):
        srcb = refs[0:IB]
        dstb = refs[IB:2 * IB]
        ewb = refs[2 * IB:3 * IB]
        rows = refs[3 * IB:3 * IB + RB]
        acc_sh = refs[3 * IB + RB]
        sems = refs[3 * IB + RB + 1:]
        isem = sems[0:IB]
        gsem = sems[IB:IB + RB]
        ssem = sems[IB + RB:IB + 2 * RB]

        cid = lax.axis_index("c")
        sid = lax.axis_index("s")
        ebase = (cid * NS + sid) * EPT

        def idx_issue(wi, j):
            base = ebase + wi * C
            pltpu.async_copy(src_hbm.at[pl.ds(base, C)], srcb[j], isem[j])
            pltpu.async_copy(dst_hbm.at[pl.ds(base, C)], dstb[j], isem[j])
            pltpu.async_copy(ew_hbm.at[pl.ds(base, C)], ewb[j], isem[j])

        def idx_wait(wi, j):
            base = ebase + wi * C
            pltpu.make_async_copy(
                src_hbm.at[pl.ds(base, C)], srcb[j], isem[j]).wait()
            pltpu.make_async_copy(
                dst_hbm.at[pl.ds(base, C)], dstb[j], isem[j]).wait()
            pltpu.make_async_copy(
                ew_hbm.at[pl.ds(base, C)], ewb[j], isem[j]).wait()

        def gather_issue(j, b):
            pltpu.async_copy(hw_hbm.at[srcb[j]], rows[b], gsem[b])

        def gather_wait(j, b):
            pltpu.make_async_copy(hw_hbm.at[srcb[j]], rows[b], gsem[b]).wait()

        def scatter_issue(j, b):
            pltpu.async_copy(rows[b], acc_sh.at[dstb[j]], ssem[b], add=True)

        def scatter_wait(j, b):
            pltpu.make_async_copy(rows[b], acc_sh.at[dstb[j]], ssem[b]).wait()

        for j in range(5):
            idx_issue(j, j)
        for j in range(3):
            idx_wait(j, j)
            gather_issue(j, j)

        pltpu.sync_copy(z_hbm.at[pl.ds(sid * RPT, RPT)],
                        acc_sh.at[pl.ds(sid * RPT, RPT)])
        plsc.subcore_barrier()

        @pl.loop(0, NWIN, step=IB)
        def _(w):
            for b in range(IB):
                wi = w + b
                rb = b % RB
                rb3 = (b + 3) % RB
                j3 = (b + 3) % IB
                j5 = (b + 5) % IB

                @pl.when(wi + 5 < NWIN)
                def _():
                    idx_issue(wi + 5, j5)

                @pl.when(wi + 3 < NWIN)
                def _():
                    @pl.when(wi >= 1)
                    def _():
                        scatter_wait(j3, rb3)
                    idx_wait(wi + 3, j3)
                    gather_issue(j3, rb3)

                gather_wait(b, rb)

                @pl.loop(0, C)
                def _(r):
                    wvec = plsc.load_gather(
                        ewb[b], [jnp.full((L,), r, jnp.int32)])
                    for cc in range(D // L):
                        sl = pl.ds(cc * L, L)
                        rows[rb][r, sl] = rows[rb][r, sl] * wvec

                scatter_issue(b, rb)

        for b in range(RB):
            scatter_wait((NWIN - RB + b) % IB, b % RB)

        plsc.subcore_barrier()
        pltpu.sync_copy(acc_sh.at[pl.ds(sid * RPT, RPT)],
                        out_hbm.at[cid, pl.ds(sid * RPT, RPT)])

    return k(hw, src, dst, ew, zeros)


def kernel(x, edge_index, edge_weight, W_enc, b_enc, W1, b1, W2, b2, W_dec, b_dec):
    pad = EPAD - E
    src = jnp.concatenate(
        [edge_index[0].astype(jnp.int32), jnp.zeros((pad,), jnp.int32)])
    dst = jnp.concatenate(
        [edge_index[1].astype(jnp.int32), jnp.zeros((pad,), jnp.int32)])
    ew = jnp.concatenate(
        [edge_weight.astype(jnp.float32), jnp.zeros((pad,), jnp.float32)])
    zeros = jnp.zeros((NPAD, D), jnp.float32)

    hw1 = _encode(x, W_enc, b_enc, W1)
    p1 = _sc_edge_pass(hw1, src, dst, ew, zeros)
    hw2 = _mid(p1, b1, W2)
    p2 = _sc_edge_pass(hw2, src, dst, ew, zeros)
    return _decode(p2, b2, W_dec, b_dec)


# speedup vs baseline: 1.5983x; 1.5983x over previous
import dataclasses
import functools

import jax
import jax.numpy as jnp
from jax import lax
from jax.experimental import pallas as pl
from jax.experimental.pallas import tpu as pltpu
from jax.experimental.pallas import tpu_sc as plsc

N = 10000
D = 128
DP = D // 2
E = 320000
NCLS = 40

NC = 2
NS = 16
NW = NC * NS
L = 16

EPAD = 327680
EPT = EPAD // NW
C = 80
NWIN = EPT // C
RB = 4
SB = 2
IB = 8
NPAD = 10240
RPT = NPAD // NS


def _encode(x, W_enc, b_enc, W1):
    def body(x_ref, we_ref, be_ref, w1_ref, o_ref):
        h = jnp.tanh(
            jnp.dot(x_ref[...], we_ref[...], preferred_element_type=jnp.float32)
            + be_ref[...]
        )
        o_ref[...] = jnp.dot(h, w1_ref[...], preferred_element_type=jnp.float32)

    return pl.pallas_call(
        body,
        out_shape=jax.ShapeDtypeStruct((N, D), jnp.float32),
    )(x, W_enc, b_enc.reshape(1, D), W1)


def _mid(parts, b, W):
    def body(p_ref, b_ref, w_ref, o_ref):
        h = jnp.tanh(p_ref[0, :N, :] + p_ref[1, :N, :] + b_ref[...])
        o_ref[...] = jnp.dot(h, w_ref[...], preferred_element_type=jnp.float32)

    return pl.pallas_call(
        body,
        out_shape=jax.ShapeDtypeStruct((N, D), jnp.float32),
    )(parts, b.reshape(1, D), W)


def _decode(parts, b2, W_dec, b_dec):
    def body(p_ref, b2_ref, wd_ref, bd_ref, o_ref):
        h = jnp.tanh(p_ref[0, :N, :] + p_ref[1, :N, :] + b2_ref[...])
        o_ref[...] = (
            jnp.dot(h, wd_ref[...], preferred_element_type=jnp.float32)
            + bd_ref[...]
        )

    return pl.pallas_call(
        body,
        out_shape=jax.ShapeDtypeStruct((N, NCLS), jnp.float32),
    )(parts, b2.reshape(1, D), W_dec, b_dec.reshape(1, NCLS))


def _pack(hw):
    a = hw.astype(jnp.bfloat16).reshape(N, D // 32, 2, 16)
    st = jnp.stack([a[:, :, 0, :], a[:, :, 1, :]], axis=-1)
    return jax.lax.bitcast_convert_type(st, jnp.int32).reshape(N, DP)


def _sc_edge_pass(hwp, src, dst, ew, zeros):
    mesh = plsc.VectorSubcoreMesh(core_axis_name="c", subcore_axis_name="s")
    cp = pltpu.CompilerParams()
    if "needs_layout_passes" in pltpu.CompilerParams.__dataclass_fields__:
        cp = dataclasses.replace(cp, needs_layout_passes=False)
    if "use_tc_tiling_on_sc" in pltpu.CompilerParams.__dataclass_fields__:
        cp = dataclasses.replace(cp, use_tc_tiling_on_sc=False)

    @functools.partial(
        pl.kernel,
        mesh=mesh,
        compiler_params=cp,
        out_type=jax.ShapeDtypeStruct((NC, NPAD, D), jnp.float32),
        scratch_types=(
            [pltpu.VMEM((C,), jnp.int32) for _ in range(IB)]
            + [pltpu.VMEM((C,), jnp.int32) for _ in range(IB)]
            + [pltpu.VMEM((C,), jnp.float32) for _ in range(IB)]
            + [pltpu.VMEM((C, DP), jnp.int32) for _ in range(RB)]
            + [pltpu.VMEM((C, D), jnp.float32) for _ in range(SB)]
            + [pltpu.VMEM_SHARED((NPAD, D), jnp.float32)]
            + [pltpu.SemaphoreType.DMA for _ in range(IB + RB + SB)]
        ),
    )
    def k(hw_hbm, src_hbm, dst_hbm, ew_hbm, z_hbm, out_hbm, *refs):
        srcb = refs[0:IB]
        dstb = refs[IB:2 * IB]
        ewb = refs[2 * IB:3 * IB]
        rows = refs[3 * IB:3 * IB + RB]
        stage = refs[3 * IB + RB:3 * IB + RB + SB]
        acc_sh = refs[3 * IB + RB + SB]
        sems = refs[3 * IB + RB + SB + 1:]
        isem = sems[0:IB]
        gsem = sems[IB:IB + RB]
        ssem = sems[IB + RB:IB + RB + SB]

        cid = lax.axis_index("c")
        sid = lax.axis_index("s")
        ebase = (cid * NS + sid) * EPT

        def idx_issue(wi, j):
            base = ebase + wi * C
            pltpu.async_copy(src_hbm.at[pl.ds(base, C)], srcb[j], isem[j])
            pltpu.async_copy(dst_hbm.at[pl.ds(base, C)], dstb[j], isem[j])
            pltpu.async_copy(ew_hbm.at[pl.ds(base, C)], ewb[j], isem[j])

        def idx_wait(wi, j):
            base = ebase + wi * C
            pltpu.make_async_copy(
                src_hbm.at[pl.ds(base, C)], srcb[j], isem[j]).wait()
            pltpu.make_async_copy(
                dst_hbm.at[pl.ds(base, C)], dstb[j], isem[j]).wait()
            pltpu.make_async_copy(
                ew_hbm.at[pl.ds(base, C)], ewb[j], isem[j]).wait()

        def gather_issue(j, b):
            pltpu.async_copy(hw_hbm.at[srcb[j]], rows[b], gsem[b])

        def gather_wait(j, b):
            pltpu.make_async_copy(hw_hbm.at[srcb[j]], rows[b], gsem[b]).wait()

        def scatter_issue(j, sb):
            pltpu.async_copy(stage[sb], acc_sh.at[dstb[j]], ssem[sb], add=True)

        def scatter_wait(j, sb):
            pltpu.make_async_copy(
                stage[sb], acc_sh.at[dstb[j]], ssem[sb]).wait()

        for j in range(5):
            idx_issue(j, j)
        for j in range(3):
            idx_wait(j, j)
            gather_issue(j, j)

        pltpu.sync_copy(z_hbm.at[pl.ds(sid * RPT, RPT)],
                        acc_sh.at[pl.ds(sid * RPT, RPT)])
        plsc.subcore_barrier()

        @pl.loop(0, NWIN, step=IB)
        def _(w):
            for b in range(IB):
                wi = w + b
                rb = b % RB
                rb3 = (b + 3) % RB
                sb = b % SB
                j3 = (b + 3) % IB
                j5 = (b + 5) % IB

                @pl.when(wi + 5 < NWIN)
                def _():
                    idx_issue(wi + 5, j5)

                @pl.when(wi + 3 < NWIN)
                def _():
                    idx_wait(wi + 3, j3)
                    gather_issue(j3, rb3)

                gather_wait(b, rb)

                @pl.when(wi >= SB)
                def _():
                    scatter_wait((b - SB) % IB, sb)

                @pl.loop(0, C)
                def _(r):
                    wvec = plsc.load_gather(
                        ewb[b], [jnp.full((L,), r, jnp.int32)])
                    for cc in range(D // 32):
                        pk = rows[rb][r, pl.ds(cc * 16, 16)]
                        bfv = plsc.bitcast(pk, jnp.bfloat16)
                        lo, hi = plsc.unpack(
                            bfv, format=plsc.PackFormat.INTERLEAVED)
                        stage[sb][r, pl.ds(cc * 32, L)] = lo * wvec
                        stage[sb][r, pl.ds(cc * 32 + L, L)] = hi * wvec

                scatter_issue(b, sb)

        for s in range(SB):
            scatter_wait((NWIN - SB + s) % IB, (NWIN - SB + s) % SB)

        plsc.subcore_barrier()
        pltpu.sync_copy(acc_sh.at[pl.ds(sid * RPT, RPT)],
                        out_hbm.at[cid, pl.ds(sid * RPT, RPT)])

    return k(hwp, src, dst, ew, zeros)


def kernel(x, edge_index, edge_weight, W_enc, b_enc, W1, b1, W2, b2, W_dec, b_dec):
    pad = EPAD - E
    src = jnp.concatenate(
        [edge_index[0].astype(jnp.int32), jnp.zeros((pad,), jnp.int32)])
    dst = jnp.concatenate(
        [edge_index[1].astype(jnp.int32), jnp.zeros((pad,), jnp.int32)])
    ew = jnp.concatenate(
        [edge_weight.astype(jnp.float32), jnp.zeros((pad,), jnp.float32)])
    zeros = jnp.zeros((NPAD, D), jnp.float32)

    hw1 = _encode(x, W_enc, b_enc, W1)
    p1 = _sc_edge_pass(_pack(hw1), src, dst, ew, zeros)
    hw2 = _mid(p1, b1, W2)
    p2 = _sc_edge_pass(_pack(hw2), src, dst, ew, zeros)
    return _decode(p2, b2, W_dec, b_dec)


# speedup vs baseline: 1.7014x; 1.0645x over previous
import dataclasses
import functools

import jax
import jax.numpy as jnp
from jax import lax
from jax.experimental import pallas as pl
from jax.experimental.pallas import tpu as pltpu
from jax.experimental.pallas import tpu_sc as plsc

N = 10000
D = 128
DP = D // 2
E = 320000
NCLS = 40

NC = 2
NS = 16
NW = NC * NS
L = 16

EPAD = 327680
EPT = EPAD // NW
C = 80
NWIN = EPT // C
RB = 4
SB = 2
IB = 8
NPAD = 10240
RPT = NPAD // NS


def _pack_tc(hw):
    u = jax.lax.bitcast_convert_type(
        hw.astype(jnp.bfloat16), jnp.uint16).astype(jnp.uint32)
    pk = [u[:, c * 32:c * 32 + 16] | (u[:, c * 32 + 16:c * 32 + 32] << 16)
          for c in range(D // 32)]
    return jnp.concatenate(pk, axis=1).astype(jnp.int32)


def _encode(x, W_enc, b_enc, W1):
    def body(x_ref, we_ref, be_ref, w1_ref, o_ref):
        h = jnp.tanh(
            jnp.dot(x_ref[...], we_ref[...], preferred_element_type=jnp.float32)
            + be_ref[...]
        )
        o_ref[...] = _pack_tc(
            jnp.dot(h, w1_ref[...], preferred_element_type=jnp.float32))

    return pl.pallas_call(
        body,
        out_shape=jax.ShapeDtypeStruct((N, DP), jnp.int32),
    )(x, W_enc, b_enc.reshape(1, D), W1)


def _mid(parts, b, W):
    def body(p_ref, b_ref, w_ref, o_ref):
        h = jnp.tanh(p_ref[0, :N, :] + p_ref[1, :N, :] + b_ref[...])
        o_ref[...] = _pack_tc(
            jnp.dot(h, w_ref[...], preferred_element_type=jnp.float32))

    return pl.pallas_call(
        body,
        out_shape=jax.ShapeDtypeStruct((N, DP), jnp.int32),
    )(parts, b.reshape(1, D), W)


def _decode(parts, b2, W_dec, b_dec):
    def body(p_ref, b2_ref, wd_ref, bd_ref, o_ref):
        h = jnp.tanh(p_ref[0, :N, :] + p_ref[1, :N, :] + b2_ref[...])
        o_ref[...] = (
            jnp.dot(h, wd_ref[...], preferred_element_type=jnp.float32)
            + bd_ref[...]
        )

    return pl.pallas_call(
        body,
        out_shape=jax.ShapeDtypeStruct((N, NCLS), jnp.float32),
    )(parts, b2.reshape(1, D), W_dec, b_dec.reshape(1, NCLS))


def _sc_edge_pass(hwp, src, dst, ew):
    mesh = plsc.VectorSubcoreMesh(core_axis_name="c", subcore_axis_name="s")
    cp = pltpu.CompilerParams()
    if "needs_layout_passes" in pltpu.CompilerParams.__dataclass_fields__:
        cp = dataclasses.replace(cp, needs_layout_passes=False)
    if "use_tc_tiling_on_sc" in pltpu.CompilerParams.__dataclass_fields__:
        cp = dataclasses.replace(cp, use_tc_tiling_on_sc=False)

    @functools.partial(
        pl.kernel,
        mesh=mesh,
        compiler_params=cp,
        out_type=jax.ShapeDtypeStruct((NC, NPAD, D), jnp.float32),
        scratch_types=(
            [pltpu.VMEM((C,), jnp.int32) for _ in range(IB)]
            + [pltpu.VMEM((C,), jnp.int32) for _ in range(IB)]
            + [pltpu.VMEM((C,), jnp.float32) for _ in range(IB)]
            + [pltpu.VMEM((C, DP), jnp.int32) for _ in range(RB)]
            + [pltpu.VMEM((C, D), jnp.float32) for _ in range(SB)]
            + [pltpu.VMEM_SHARED((NPAD, D), jnp.float32)]
            + [pltpu.SemaphoreType.DMA for _ in range(IB + RB + SB)]
        ),
    )
    def k(hw_hbm, src_hbm, dst_hbm, ew_hbm, out_hbm, *refs):
        srcb = refs[0:IB]
        dstb = refs[IB:2 * IB]
        ewb = refs[2 * IB:3 * IB]
        rows = refs[3 * IB:3 * IB + RB]
        stage = refs[3 * IB + RB:3 * IB + RB + SB]
        acc_sh = refs[3 * IB + RB + SB]
        sems = refs[3 * IB + RB + SB + 1:]
        isem = sems[0:IB]
        gsem = sems[IB:IB + RB]
        ssem = sems[IB + RB:IB + RB + SB]

        cid = lax.axis_index("c")
        sid = lax.axis_index("s")
        ebase = (cid * NS + sid) * EPT

        def idx_issue(wi, j):
            base = ebase + wi * C
            pltpu.async_copy(src_hbm.at[pl.ds(base, C)], srcb[j], isem[j])
            pltpu.async_copy(dst_hbm.at[pl.ds(base, C)], dstb[j], isem[j])
            pltpu.async_copy(ew_hbm.at[pl.ds(base, C)], ewb[j], isem[j])

        def idx_wait(wi, j):
            base = ebase + wi * C
            pltpu.make_async_copy(
                src_hbm.at[pl.ds(base, C)], srcb[j], isem[j]).wait()
            pltpu.make_async_copy(
                dst_hbm.at[pl.ds(base, C)], dstb[j], isem[j]).wait()
            pltpu.make_async_copy(
                ew_hbm.at[pl.ds(base, C)], ewb[j], isem[j]).wait()

        def gather_issue(j, b):
            pltpu.async_copy(hw_hbm.at[srcb[j]], rows[b], gsem[b])

        def gather_wait(j, b):
            pltpu.make_async_copy(hw_hbm.at[srcb[j]], rows[b], gsem[b]).wait()

        def scatter_issue(j, sb):
            pltpu.async_copy(stage[sb], acc_sh.at[dstb[j]], ssem[sb], add=True)

        def scatter_wait(j, sb):
            pltpu.make_async_copy(
                stage[sb], acc_sh.at[dstb[j]], ssem[sb]).wait()

        for j in range(5):
            idx_issue(j, j)
        for j in range(3):
            idx_wait(j, j)
            gather_issue(j, j)

        @pl.loop(0, C)
        def _(r):
            for cc in range(D // L):
                stage[0][r, pl.ds(cc * L, L)] = jnp.zeros((L,), jnp.float32)
        for t in range(RPT // C):
            pltpu.sync_copy(stage[0],
                            acc_sh.at[pl.ds(sid * RPT + t * C, C)])
        plsc.subcore_barrier()

        @pl.loop(0, NWIN, step=IB)
        def _(w):
            for b in range(IB):
                wi = w + b
                rb = b % RB
                rb3 = (b + 3) % RB
                sb = b % SB
                j3 = (b + 3) % IB
                j5 = (b + 5) % IB

                @pl.when(wi + 5 < NWIN)
                def _():
                    idx_issue(wi + 5, j5)

                @pl.when(wi + 3 < NWIN)
                def _():
                    idx_wait(wi + 3, j3)
                    gather_issue(j3, rb3)

                gather_wait(b, rb)

                @pl.when(wi >= SB)
                def _():
                    scatter_wait((b - SB) % IB, sb)

                @pl.loop(0, C)
                def _(r):
                    wvec = plsc.load_gather(
                        ewb[b], [jnp.full((L,), r, jnp.int32)])
                    for cc in range(D // 32):
                        pk = rows[rb][r, pl.ds(cc * 16, 16)]
                        bfv = plsc.bitcast(pk, jnp.bfloat16)
                        lo, hi = plsc.unpack(
                            bfv, format=plsc.PackFormat.INTERLEAVED)
                        stage[sb][r, pl.ds(cc * 32, L)] = lo * wvec
                        stage[sb][r, pl.ds(cc * 32 + L, L)] = hi * wvec

                scatter_issue(b, sb)

        for s in range(SB):
            scatter_wait((NWIN - SB + s) % IB, (NWIN - SB + s) % SB)

        plsc.subcore_barrier()
        pltpu.sync_copy(acc_sh.at[pl.ds(sid * RPT, RPT)],
                        out_hbm.at[cid, pl.ds(sid * RPT, RPT)])

    return k(hwp, src, dst, ew)


def kernel(x, edge_index, edge_weight, W_enc, b_enc, W1, b1, W2, b2, W_dec, b_dec):
    pad = EPAD - E
    src = jnp.concatenate(
        [edge_index[0].astype(jnp.int32), jnp.zeros((pad,), jnp.int32)])
    dst = jnp.concatenate(
        [edge_index[1].astype(jnp.int32), jnp.zeros((pad,), jnp.int32)])
    ew = jnp.concatenate(
        [edge_weight.astype(jnp.float32), jnp.zeros((pad,), jnp.float32)])

    hwp1 = _encode(x, W_enc, b_enc, W1)
    p1 = _sc_edge_pass(hwp1, src, dst, ew)
    hwp2 = _mid(p1, b1, W2)
    p2 = _sc_edge_pass(hwp2, src, dst, ew)
    return _decode(p2, b2, W_dec, b_dec)


# speedup vs baseline: 1.9412x; 1.1409x over previous
import dataclasses
import functools

import jax
import jax.numpy as jnp
from jax import lax
from jax.experimental import pallas as pl
from jax.experimental.pallas import tpu as pltpu
from jax.experimental.pallas import tpu_sc as plsc

N = 10000
D = 128
DP = D // 2
E = 320000
NCLS = 40

NC = 2
NS = 16
NW = NC * NS
L = 16

EPAD = 327680
EPT = EPAD // NW
C = 80
NWIN = EPT // C
RB = 4
SB = 2
IB = 8
NPAD = 10240
RPT = NPAD // NS


def _pack_tc(hw):
    u = jax.lax.bitcast_convert_type(
        hw.astype(jnp.bfloat16), jnp.uint16).astype(jnp.uint32)
    pk = [u[:, c * 32:c * 32 + 16] | (u[:, c * 32 + 16:c * 32 + 32] << 16)
          for c in range(D // 32)]
    return jnp.concatenate(pk, axis=1).astype(jnp.int32)


def _encode(x, W_enc, b_enc, W1):
    def body(x_ref, we_ref, be_ref, w1_ref, o_ref):
        h = jnp.tanh(
            jnp.dot(x_ref[...], we_ref[...], preferred_element_type=jnp.float32)
            + be_ref[...]
        )
        o_ref[...] = _pack_tc(
            jnp.dot(h, w1_ref[...], preferred_element_type=jnp.float32))

    return pl.pallas_call(
        body,
        out_shape=jax.ShapeDtypeStruct((N, DP), jnp.int32),
    )(x, W_enc, b_enc.reshape(1, D), W1)


def _mid(parts, b, W):
    def body(p_ref, b_ref, w_ref, o_ref):
        h = jnp.tanh(p_ref[0, :N, :] + p_ref[1, :N, :] + b_ref[...])
        o_ref[...] = _pack_tc(
            jnp.dot(h, w_ref[...], preferred_element_type=jnp.float32))

    return pl.pallas_call(
        body,
        out_shape=jax.ShapeDtypeStruct((N, DP), jnp.int32),
    )(parts, b.reshape(1, D), W)


def _decode(parts, b2, W_dec, b_dec):
    def body(p_ref, b2_ref, wd_ref, bd_ref, o_ref):
        h = jnp.tanh(p_ref[0, :N, :] + p_ref[1, :N, :] + b2_ref[...])
        o_ref[...] = (
            jnp.dot(h, wd_ref[...], preferred_element_type=jnp.float32)
            + bd_ref[...]
        )

    return pl.pallas_call(
        body,
        out_shape=jax.ShapeDtypeStruct((N, NCLS), jnp.float32),
    )(parts, b2.reshape(1, D), W_dec, b_dec.reshape(1, NCLS))


def _sc_edge_pass(hwp, src, dst, ew):
    mesh = plsc.VectorSubcoreMesh(core_axis_name="c", subcore_axis_name="s")
    cp = pltpu.CompilerParams()
    if "needs_layout_passes" in pltpu.CompilerParams.__dataclass_fields__:
        cp = dataclasses.replace(cp, needs_layout_passes=False)
    if "use_tc_tiling_on_sc" in pltpu.CompilerParams.__dataclass_fields__:
        cp = dataclasses.replace(cp, use_tc_tiling_on_sc=False)

    @functools.partial(
        pl.kernel,
        mesh=mesh,
        compiler_params=cp,
        out_type=jax.ShapeDtypeStruct((NC, NPAD, D), jnp.float32),
        scratch_types=(
            [pltpu.VMEM((C,), jnp.int32) for _ in range(IB)]
            + [pltpu.VMEM((C,), jnp.int32) for _ in range(IB)]
            + [pltpu.VMEM((C,), jnp.float32) for _ in range(IB)]
            + [pltpu.VMEM((C, DP), jnp.int32) for _ in range(RB)]
            + [pltpu.VMEM((C, D), jnp.float32) for _ in range(SB)]
            + [pltpu.VMEM_SHARED((NPAD, D), jnp.float32)]
            + [pltpu.SemaphoreType.DMA for _ in range(IB + RB + SB)]
        ),
    )
    def k(hw_hbm, src_hbm, dst_hbm, ew_hbm, out_hbm, *refs):
        srcb = refs[0:IB]
        dstb = refs[IB:2 * IB]
        ewb = refs[2 * IB:3 * IB]
        rows = refs[3 * IB:3 * IB + RB]
        stage = refs[3 * IB + RB:3 * IB + RB + SB]
        acc_sh = refs[3 * IB + RB + SB]
        sems = refs[3 * IB + RB + SB + 1:]
        isem = sems[0:IB]
        gsem = sems[IB:IB + RB]
        ssem = sems[IB + RB:IB + RB + SB]

        cid = lax.axis_index("c")
        sid = lax.axis_index("s")
        ebase = (cid * NS + sid) * EPT

        def idx_issue(wi, j):
            base = ebase + wi * C
            pltpu.async_copy(src_hbm.at[pl.ds(base, C)], srcb[j], isem[j])
            pltpu.async_copy(dst_hbm.at[pl.ds(base, C)], dstb[j], isem[j])
            pltpu.async_copy(ew_hbm.at[pl.ds(base, C)], ewb[j], isem[j])

        def idx_wait(wi, j):
            base = ebase + wi * C
            pltpu.make_async_copy(
                src_hbm.at[pl.ds(base, C)], srcb[j], isem[j]).wait()
            pltpu.make_async_copy(
                dst_hbm.at[pl.ds(base, C)], dstb[j], isem[j]).wait()
            pltpu.make_async_copy(
                ew_hbm.at[pl.ds(base, C)], ewb[j], isem[j]).wait()

        def gather_issue(j, b):
            pltpu.async_copy(hw_hbm.at[srcb[j]], rows[b], gsem[b])

        def gather_wait(j, b):
            pltpu.make_async_copy(hw_hbm.at[srcb[j]], rows[b], gsem[b]).wait()

        def scatter_issue(j, sb):
            pltpu.async_copy(stage[sb], acc_sh.at[dstb[j]], ssem[sb], add=True)

        def scatter_wait(j, sb):
            pltpu.make_async_copy(
                stage[sb], acc_sh.at[dstb[j]], ssem[sb]).wait()

        for j in range(5):
            idx_issue(j, j)
        for j in range(3):
            idx_wait(j, j)
            gather_issue(j, j)

        @pl.loop(0, C)
        def _(r):
            for cc in range(D // L):
                stage[0][r, pl.ds(cc * L, L)] = jnp.zeros((L,), jnp.float32)
        for t in range(RPT // C):
            pltpu.sync_copy(stage[0],
                            acc_sh.at[pl.ds(sid * RPT + t * C, C)])
        plsc.subcore_barrier()

        @pl.loop(0, NWIN, step=IB)
        def _(w):
            for b in range(IB):
                wi = w + b
                rb = b % RB
                rb3 = (b + 3) % RB
                sb = b % SB
                j3 = (b + 3) % IB
                j5 = (b + 5) % IB

                @pl.when(wi + 5 < NWIN)
                def _():
                    idx_issue(wi + 5, j5)

                @pl.when(wi + 3 < NWIN)
                def _():
                    idx_wait(wi + 3, j3)
                    gather_issue(j3, rb3)

                gather_wait(b, rb)

                @pl.when(wi >= SB)
                def _():
                    scatter_wait((b - SB) % IB, sb)

                @plsc.parallel_loop(0, C, unroll=2)
                def _(r):
                    wvec = plsc.load_gather(
                        ewb[b], [jnp.full((L,), r, jnp.int32)])
                    for cc in range(D // 32):
                        pk = rows[rb][r, pl.ds(cc * 16, 16)]
                        bfv = plsc.bitcast(pk, jnp.bfloat16)
                        lo, hi = plsc.unpack(
                            bfv, format=plsc.PackFormat.INTERLEAVED)
                        stage[sb][r, pl.ds(cc * 32, L)] = lo * wvec
                        stage[sb][r, pl.ds(cc * 32 + L, L)] = hi * wvec

                scatter_issue(b, sb)

        for s in range(SB):
            scatter_wait((NWIN - SB + s) % IB, (NWIN - SB + s) % SB)

        plsc.subcore_barrier()
        pltpu.sync_copy(acc_sh.at[pl.ds(sid * RPT, RPT)],
                        out_hbm.at[cid, pl.ds(sid * RPT, RPT)])

    return k(hwp, src, dst, ew)


def kernel(x, edge_index, edge_weight, W_enc, b_enc, W1, b1, W2, b2, W_dec, b_dec):
    pad = EPAD - E
    src = jnp.concatenate(
        [edge_index[0].astype(jnp.int32), jnp.zeros((pad,), jnp.int32)])
    dst = jnp.concatenate(
        [edge_index[1].astype(jnp.int32), jnp.zeros((pad,), jnp.int32)])
    ew = jnp.concatenate(
        [edge_weight.astype(jnp.float32), jnp.zeros((pad,), jnp.float32)])

    hwp1 = _encode(x, W_enc, b_enc, W1)
    p1 = _sc_edge_pass(hwp1, src, dst, ew)
    hwp2 = _mid(p1, b1, W2)
    p2 = _sc_edge_pass(hwp2, src, dst, ew)
    return _decode(p2, b2, W_dec, b_dec)
